# baseline (speedup 1.0000x reference)
import functools

import jax
import jax.numpy as jnp
from jax import lax
from jax.experimental import pallas as pl
from jax.experimental.pallas import tpu as pltpu
from jax.experimental.pallas import tpu_sc as plsc

_L = 16
_WIN = 512
_WAVE = 64


@functools.cache
def _build(B, V, D, dtype):
    info = plsc.get_sparse_core_info()
    NC, NS = info.num_cores, info.num_subcores
    NW = NC * NS
    nwin = (V + _WIN - 1) // _WIN
    mesh = plsc.VectorSubcoreMesh(core_axis_name="c", subcore_axis_name="s")

    @functools.partial(
        pl.kernel,
        mesh=mesh,
        compiler_params=pltpu.CompilerParams(needs_layout_passes=False),
        out_type=jax.ShapeDtypeStruct((B, D), dtype),
        scratch_types=[
            pltpu.VMEM((B,), jnp.int32),
            pltpu.VMEM((B + _L,), jnp.int32),
            pltpu.VMEM((B + _L,), jnp.int32),
            pltpu.VMEM((2 * D, _WIN), dtype),
            pltpu.VMEM((_WAVE, D), dtype),
            pltpu.SemaphoreType.DMA,
            pltpu.SemaphoreType.DMA,
            pltpu.SemaphoreType.DMA,
        ],
    )
    def k(idx_hbm, tabT_hbm, tail_hbm, out_hbm, idx_v, pos_v, spos_v,
          win_v, stage_v, wsem, rsem, tsem):
        wid = lax.axis_index("s") * NC + lax.axis_index("c")
        wlo = (nwin * wid) // NW
        whi = (nwin * (wid + 1)) // NW
        lo = wlo * _WIN
        hi = whi * _WIN
        lanes = lax.iota(jnp.int32, _L)

        pltpu.sync_copy(idx_hbm, idx_v)

        def scan1(c, cursor):
            vec = idx_v[pl.ds(c * _L, _L)]
            mask = (vec >= lo) & (vec < hi)
            skey = jnp.where(mask, lanes, _L + lanes)
            _, payload = plsc.sort_key_val(skey, lanes + c * _L)
            pos_v[pl.ds(cursor, _L)] = payload
            cnt = plsc.all_reduce_population_count(mask)
            return cursor + cnt[0]

        mcount = lax.fori_loop(0, B // _L, scan1, jnp.int32(0),
                               unroll=False)
        mchunks = (mcount + _L - 1) // _L

        last_start = (nwin - 1) * _WIN
        last_len = V - last_start

        def fetch_win(win):
            par = (win - wlo) % 2

            @pl.when(win < nwin - 1)
            def _():
                s = pl.multiple_of(win * _WIN, _WIN)
                pltpu.async_copy(
                    tabT_hbm.at[:, pl.ds(s, _WIN)],
                    win_v.at[pl.ds(par * D, D), pl.ds(0, _WIN)], wsem)

        @pl.when((whi > wlo) & (wlo < nwin - 1))
        def _():
            fetch_win(wlo)

        def per_window(win, _):
            par = (win - wlo) % 2

            @pl.when(win < nwin - 1)
            def _():
                pltpu.make_async_copy(
                    tabT_hbm.at[:, pl.ds(0, _WIN)],
                    win_v.at[pl.ds(par * D, D), pl.ds(0, _WIN)],
                    wsem).wait()

            @pl.when(win + 1 < whi)
            def _():
                fetch_win(win + 1)

            wstart = win * _WIN
            klo = win * _WIN
            khi = klo + _WIN

            def scan2(m, cursor):
                valid = (m * _L + lanes) < mcount
                pvec = pos_v[pl.ds(m * _L, _L)]
                kvec = plsc.load_gather(
                    idx_v, [jnp.clip(pvec, 0, B - 1)])
                wmask = valid & (kvec >= klo) & (kvec < khi)
                skey = jnp.where(wmask, lanes, _L + lanes)
                _, payload = plsc.sort_key_val(skey, pvec)
                spos_v[pl.ds(cursor, _L)] = payload
                cnt = plsc.all_reduce_population_count(wmask)
                return cursor + cnt[0]

            scount = lax.fori_loop(0, mchunks, scan2, jnp.int32(0),
                                   unroll=False)

            def per_wave(v, _):
                wbase = v * _WAVE
                nhere = jnp.minimum(scount - wbase, _WAVE)
                for t in range(_WAVE // _L):
                    svec = spos_v[pl.ds(wbase + t * _L, _L)]
                    kvec = plsc.load_gather(
                        idx_v, [jnp.clip(svec, 0, B - 1)])

                    @pl.when(win < nwin - 1)
                    def _(svec=svec, kvec=kvec, t=t):
                        off = jnp.clip(kvec - wstart, 0, _WIN - 1)
                        for d in range(D):
                            vals = plsc.load_gather(
                                win_v, [jnp.full((_L,), par * D + d,
                                                 jnp.int32), off])
                            plsc.store_scatter(
                                stage_v, [lanes + t * _L,
                                          jnp.full((_L,), d, jnp.int32)],
                                vals)

                    @pl.when(win == nwin - 1)
                    def _(svec=svec, kvec=kvec, t=t):
                        for j in range(_L):
                            mm = wbase + t * _L + j

                            @pl.when(mm < scount)
                            def _(j=j, t=t, kvec=kvec):
                                koff = kvec[j] - last_start
                                pltpu.async_copy(
                                    tail_hbm.at[pl.ds(koff, 1)],
                                    stage_v.at[pl.ds(t * _L + j, 1)],
                                    tsem)

                @pl.when(win == nwin - 1)
                def _():
                    def tdrain(u, _):
                        pltpu.make_async_copy(
                            tail_hbm.at[pl.ds(0, 1)],
                            stage_v.at[pl.ds(0, 1)], tsem).wait()
                        return ()

                    lax.fori_loop(0, nhere, tdrain, (), unroll=False)
                for t in range(_WAVE // _L):
                    svec = spos_v[pl.ds(wbase + t * _L, _L)]
                    for j in range(_L):
                        mm = wbase + t * _L + j

                        @pl.when(mm < scount)
                        def _(t=t, j=j, svec=svec):
                            p = svec[j]
                            pltpu.async_copy(
                                stage_v.at[pl.ds(t * _L + j, 1)],
                                out_hbm.at[pl.ds(p, 1)], rsem)

                def drain(u, _):
                    pltpu.make_async_copy(
                        stage_v.at[pl.ds(0, 1)],
                        out_hbm.at[pl.ds(0, 1)], rsem).wait()
                    return ()

                lax.fori_loop(0, nhere, drain, (), unroll=False)
                return ()

            nwaves = (scount + _WAVE - 1) // _WAVE
            lax.fori_loop(0, nwaves, per_wave, (), unroll=False)
            return ()

        lax.fori_loop(wlo, whi, per_window, (), unroll=False)

    return k


def kernel(users, items, user_table, item_table):
    B = users.shape[0]
    V, D = user_table.shape
    nwin = (V + _WIN - 1) // _WIN
    last_start = (nwin - 1) * _WIN
    k = _build(B, V, D, user_table.dtype)
    users_embs = k(users.astype(jnp.int32), user_table.T,
                   user_table[last_start:])
    items_embs = k(items.astype(jnp.int32), item_table.T,
                   item_table[last_start:])
    return (users_embs, items_embs)


# speedup vs baseline: 1.5444x; 1.5444x over previous
import functools

import jax
import jax.numpy as jnp
from jax import lax
from jax.experimental import pallas as pl
from jax.experimental.pallas import tpu as pltpu
from jax.experimental.pallas import tpu_sc as plsc

_L = 16


@functools.cache
def _build(B, D, dtype):
    info = plsc.get_sparse_core_info()
    NC, NS = info.num_cores, info.num_subcores
    NW = NC * NS
    b_per_w = B // NW
    half = b_per_w // 2
    mesh = plsc.VectorSubcoreMesh(core_axis_name="c", subcore_axis_name="s")

    @functools.partial(
        pl.kernel,
        mesh=mesh,
        compiler_params=pltpu.CompilerParams(needs_layout_passes=False),
        out_type=jax.ShapeDtypeStruct((D, B), dtype),
        scratch_types=[
            pltpu.VMEM((b_per_w,), jnp.int32),
            pltpu.VMEM((half, D), dtype),
            pltpu.VMEM((D, half), dtype),
            pltpu.SemaphoreType.DMA,
        ],
    )
    def k(idx_hbm, tab_hbm, out_hbm, idx_v, rows_v, trans_v, sem):
        lanes = lax.iota(jnp.int32, _L)
        wid = lax.axis_index("s") * NC + lax.axis_index("c")
        base = wid * b_per_w
        pltpu.sync_copy(idx_hbm.at[pl.ds(base, b_per_w)], idx_v)

        for h in range(2):
            off = h * half

            def fetch(c, _):
                vec = idx_v[pl.ds(off + c * _L, _L)]
                for j in range(_L):
                    i = c * _L + j
                    r = vec[j]
                    pltpu.async_copy(tab_hbm.at[pl.ds(r, 1)],
                                     rows_v.at[pl.ds(i, 1)], sem)
                return ()

            lax.fori_loop(0, half // _L, fetch, (), unroll=False)
            pltpu.make_async_copy(tab_hbm.at[pl.ds(0, half)], rows_v,
                                  sem).wait()

            def transpose(cc, _):
                rows16 = cc * _L + lanes
                for d in range(D):
                    vals = plsc.load_gather(
                        rows_v, [rows16, jnp.full((_L,), d, jnp.int32)])
                    plsc.store_scatter(
                        trans_v, [jnp.full((_L,), d, jnp.int32), rows16],
                        vals)
                return ()

            lax.fori_loop(0, half // _L, transpose, (), unroll=False)
            s = pl.multiple_of(base + off, half)
            pltpu.sync_copy(trans_v, out_hbm.at[:, pl.ds(s, half)])

    return k


def kernel(users, items, user_table, item_table):
    B = users.shape[0]
    D = user_table.shape[1]
    k = _build(B, D, user_table.dtype)
    users_embs = k(users.astype(jnp.int32), user_table)
    items_embs = k(items.astype(jnp.int32), item_table)
    return (users_embs.T, items_embs.T)


# speedup vs baseline: 1.8068x; 1.1699x over previous
import functools

import jax
import jax.numpy as jnp
from jax import lax
from jax.experimental import pallas as pl
from jax.experimental.pallas import tpu as pltpu
from jax.experimental.pallas import tpu_sc as plsc

_L = 16


@functools.cache
def _build(B, D, dtype):
    info = plsc.get_sparse_core_info()
    NC, NS = info.num_cores, info.num_subcores
    NW = NC * NS
    b_per_w = B // NW
    half = b_per_w // 2
    mesh = plsc.VectorSubcoreMesh(core_axis_name="c", subcore_axis_name="s")

    @functools.partial(
        pl.kernel,
        mesh=mesh,
        out_type=jax.ShapeDtypeStruct((B, D), dtype),
        scratch_types=[
            pltpu.VMEM((b_per_w,), jnp.int32),
            pltpu.VMEM((half, D), dtype),
            pltpu.SemaphoreType.DMA,
        ],
    )
    def k(idx_hbm, tab_hbm, out_hbm, idx_v, rows_v, sem):
        wid = lax.axis_index("s") * NC + lax.axis_index("c")
        base = wid * b_per_w
        pltpu.sync_copy(idx_hbm.at[pl.ds(base, b_per_w)], idx_v)

        for h in range(2):
            off = h * half

            def fetch(c, _):
                vec = idx_v[pl.ds(off + c * _L, _L)]
                for j in range(_L):
                    i = c * _L + j
                    r = vec[j]
                    pltpu.async_copy(tab_hbm.at[pl.ds(r, 1)],
                                     rows_v.at[pl.ds(i, 1)], sem)
                return ()

            lax.fori_loop(0, half // _L, fetch, (), unroll=False)
            pltpu.make_async_copy(tab_hbm.at[pl.ds(0, half)], rows_v,
                                  sem).wait()
            pltpu.sync_copy(rows_v, out_hbm.at[pl.ds(base + off, half)])

    return k


def kernel(users, items, user_table, item_table):
    B = users.shape[0]
    D = user_table.shape[1]
    k = _build(B, D, user_table.dtype)
    users_embs = k(users.astype(jnp.int32), user_table)
    items_embs = k(items.astype(jnp.int32), item_table)
    return (users_embs, items_embs)


# speedup vs baseline: 1.8146x; 1.0043x over previous
import functools

import jax
import jax.numpy as jnp
from jax import lax
from jax.experimental import pallas as pl
from jax.experimental.pallas import tpu as pltpu
from jax.experimental.pallas import tpu_sc as plsc

_L = 16


@functools.cache
def _build(B, D, dtype):
    info = plsc.get_sparse_core_info()
    NC, NS = info.num_cores, info.num_subcores
    NW = NC * NS
    b_per_w = B // NW
    mesh = plsc.VectorSubcoreMesh(core_axis_name="c", subcore_axis_name="s")

    @functools.partial(
        pl.kernel,
        mesh=mesh,
        out_type=jax.ShapeDtypeStruct((B, D), dtype),
        scratch_types=[
            pltpu.VMEM((b_per_w,), jnp.int32),
            pltpu.VMEM((b_per_w, D), dtype),
            pltpu.SemaphoreType.DMA,
        ],
    )
    def k(idx_hbm, tab_hbm, out_hbm, idx_v, rows_v, sem):
        wid = lax.axis_index("s") * NC + lax.axis_index("c")
        base = wid * b_per_w
        pltpu.sync_copy(idx_hbm.at[pl.ds(base, b_per_w)], idx_v)

        def fetch(c, _):
            vec = idx_v[pl.ds(c * _L, _L)]
            for j in range(_L):
                i = c * _L + j
                r = vec[j]
                pltpu.async_copy(tab_hbm.at[pl.ds(r, 1)],
                                 rows_v.at[pl.ds(i, 1)], sem)
            return ()

        lax.fori_loop(0, b_per_w // _L, fetch, (), unroll=False)
        pltpu.make_async_copy(tab_hbm.at[pl.ds(0, b_per_w)], rows_v,
                              sem).wait()
        pltpu.sync_copy(rows_v, out_hbm.at[pl.ds(base, b_per_w)])

    return k


def kernel(users, items, user_table, item_table):
    B = users.shape[0]
    D = user_table.shape[1]
    k = _build(B, D, user_table.dtype)
    users_embs = k(users.astype(jnp.int32), user_table)
    items_embs = k(items.astype(jnp.int32), item_table)
    return (users_embs, items_embs)
